# baseline (speedup 1.0000x reference)
import functools

import jax
import jax.numpy as jnp
from jax import lax
from jax.experimental import pallas as pl
from jax.experimental.pallas import tpu as pltpu
from jax.experimental.pallas import tpu_sc as plsc

NUM_SAMPLES = 32
T = 128
CLIPS = 3
H = 224
W = 224
NC = 2
NS = 16

_MESH = plsc.VectorSubcoreMesh(
    core_axis_name="c", subcore_axis_name="s", num_cores=NC, num_subcores=NS
)


@functools.partial(
    pl.kernel,
    out_type=jax.ShapeDtypeStruct((CLIPS, NUM_SAMPLES, H, W), jnp.float32),
    mesh=_MESH,
    scratch_types=[pltpu.SemaphoreType.DMA],
)
def _sc_subsample(x_hbm, out_hbm, sem):
    cid = lax.axis_index("c")
    sid = lax.axis_index("s")
    wid = sid * NC + cid
    tsrc = (wid * (T - 1)) // (NUM_SAMPLES - 1)

    dmas = [
        pltpu.async_copy(x_hbm.at[clip, tsrc], out_hbm.at[clip, wid], sem)
        for clip in range(CLIPS)
    ]
    for dma in dmas:
        dma.wait()


def kernel(x):
    return _sc_subsample(x)


# speedup vs baseline: 6.8904x; 6.8904x over previous
import functools

import jax
import jax.numpy as jnp
from jax import lax
from jax.experimental import pallas as pl
from jax.experimental.pallas import tpu as pltpu
from jax.experimental.pallas import tpu_sc as plsc

NUM_SAMPLES = 32
T = 128
CLIPS = 3
H = 224
W = 224
NC = 2
NS = 16

_MESH = plsc.VectorSubcoreMesh(
    core_axis_name="c", subcore_axis_name="s", num_cores=NC, num_subcores=NS
)


@functools.partial(
    pl.kernel,
    out_type=jax.ShapeDtypeStruct((CLIPS, NUM_SAMPLES, H, W), jnp.float32),
    mesh=_MESH,
    scratch_types=[
        pltpu.VMEM((H, W), jnp.float32),
        pltpu.VMEM((H, W), jnp.float32),
        pltpu.SemaphoreType.DMA,
        pltpu.SemaphoreType.DMA,
        pltpu.SemaphoreType.DMA,
        pltpu.SemaphoreType.DMA,
    ],
)
def _sc_subsample(x_hbm, out_hbm, buf0, buf1, in0, in1, out0, out1):
    cid = lax.axis_index("c")
    sid = lax.axis_index("s")
    wid = sid * NC + cid
    tsrc = (wid * (T - 1)) // (NUM_SAMPLES - 1)

    bufs = (buf0, buf1)
    in_sems = (in0, in1)
    out_sems = (out0, out1)

    in_dmas = [
        pltpu.async_copy(x_hbm.at[0, tsrc], buf0, in0),
        pltpu.async_copy(x_hbm.at[1, tsrc], buf1, in1),
    ]
    out_dmas = [None, None]

    for clip in range(CLIPS):
        slot = clip % 2
        in_dmas[slot].wait()
        out_dmas[slot] = pltpu.async_copy(
            bufs[slot], out_hbm.at[clip, wid], out_sems[slot]
        )
        nxt = clip + 2
        if nxt < CLIPS:
            out_dmas[slot].wait()
            out_dmas[slot] = None
            in_dmas[slot] = pltpu.async_copy(
                x_hbm.at[nxt, tsrc], bufs[slot], in_sems[slot]
            )

    for slot in range(2):
        if out_dmas[slot] is not None:
            out_dmas[slot].wait()


def kernel(x):
    return _sc_subsample(x)
